# Initial kernel scaffold; baseline (speedup 1.0000x reference)
import functools

import jax
import jax.numpy as jnp
from jax import lax
from jax.experimental import pallas as pl
from jax.experimental.pallas import tpu as pltpu
from jax.experimental.pallas import tpu_sc as plsc

_VOCAB = 1000000
_RANK = 32
_DIM = 64
_BATCH = 16384
_HIST = 50
_TOTAL = _BATCH * _HIST

_NC = 2
_NS = 16
_NW = _NC * _NS
_PER_W = _TOTAL // _NW
_CHUNK = 128
_NCH = _PER_W // _CHUNK


def _sc_gather(idx_hbm, table_hbm, out_hbm, idx_v, rows_v, sem):
    wid = lax.axis_index("s") * _NC + lax.axis_index("c")
    pltpu.sync_copy(idx_hbm.at[wid], idx_v)

    def body(j, carry):
        pltpu.async_copy(table_hbm.at[idx_v.at[j]], rows_v, sem).wait()
        base = wid * _PER_W + j * _CHUNK
        pltpu.sync_copy(rows_v, out_hbm.at[pl.ds(base, _CHUNK)])
        return carry

    lax.fori_loop(0, _NCH, body, 0, unroll=False)


_sc_gather_call = functools.partial(
    pl.kernel,
    out_type=jax.ShapeDtypeStruct((_TOTAL, _RANK), jnp.float32),
    mesh=plsc.VectorSubcoreMesh(core_axis_name="c", subcore_axis_name="s"),
    scratch_types=[
        pltpu.VMEM((_NCH, _CHUNK), jnp.int32),
        pltpu.VMEM((_CHUNK, _RANK), jnp.float32),
        pltpu.SemaphoreType.DMA,
    ],
)(_sc_gather)


def _tc_matmul_body(emb_ref, b_ref, out_ref):
    out_ref[...] = jnp.dot(
        emb_ref[...], b_ref[...], preferred_element_type=jnp.float32
    )


_BM = 4096


def _tc_matmul(emb, B):
    grid = (_TOTAL // _BM,)
    return pl.pallas_call(
        _tc_matmul_body,
        grid=grid,
        in_specs=[
            pl.BlockSpec((_BM, _RANK), lambda i: (i, 0)),
            pl.BlockSpec((_RANK, _DIM), lambda i: (0, 0)),
        ],
        out_specs=pl.BlockSpec((_BM, _DIM), lambda i: (i, 0)),
        out_shape=jax.ShapeDtypeStruct((_TOTAL, _DIM), jnp.float32),
        compiler_params=pltpu.CompilerParams(
            dimension_semantics=("arbitrary",),
        ),
    )(emb, B)


def kernel(token_ids, A, B):
    idx = token_ids.reshape(_NW, _NCH, _CHUNK).astype(jnp.int32)
    emb = _sc_gather_call(idx, A)
    out = _tc_matmul(emb, B)
    return out.reshape(_BATCH, _HIST, _DIM)


# speedup vs baseline: 11.1269x; 11.1269x over previous
import functools

import jax
import jax.numpy as jnp
from jax import lax
from jax.experimental import pallas as pl
from jax.experimental.pallas import tpu as pltpu
from jax.experimental.pallas import tpu_sc as plsc

_VOCAB = 1000000
_RANK = 32
_DIM = 64
_BATCH = 16384
_HIST = 50
_TOTAL = _BATCH * _HIST

_NC = 2
_NS = 16
_NW = _NC * _NS
_PER_W = _TOTAL // _NW
_CHUNK = 128
_NCH = _PER_W // _CHUNK


def _sc_gather(idx_hbm, table_hbm, out_hbm, idx_v, rows_v, sem):
    wid = lax.axis_index("s") * _NC + lax.axis_index("c")
    pltpu.sync_copy(idx_hbm.at[wid], idx_v)

    def body(j, carry):
        pltpu.async_copy(table_hbm.at[idx_v.at[j]], rows_v, sem).wait()
        base = wid * _PER_W + j * _CHUNK
        pltpu.sync_copy(rows_v, out_hbm.at[pl.ds(base, _CHUNK)])
        return carry

    lax.fori_loop(0, _NCH, body, 0, unroll=False)


_sc_gather_call = functools.partial(
    pl.kernel,
    out_type=jax.ShapeDtypeStruct((_TOTAL, _RANK), jnp.float32),
    mesh=plsc.VectorSubcoreMesh(core_axis_name="c", subcore_axis_name="s"),
    scratch_types=[
        pltpu.VMEM((_NCH, _CHUNK), jnp.int32),
        pltpu.VMEM((_CHUNK, _RANK), jnp.float32),
        pltpu.SemaphoreType.DMA,
    ],
    compiler_params=pltpu.CompilerParams(use_tc_tiling_on_sc=False),
)(_sc_gather)


def _tc_matmul_body(emb_ref, b_ref, out_ref):
    out_ref[...] = jnp.dot(
        emb_ref[...], b_ref[...], preferred_element_type=jnp.float32
    )


_BM = 4096


def _tc_matmul(emb, B):
    grid = (_TOTAL // _BM,)
    return pl.pallas_call(
        _tc_matmul_body,
        grid=grid,
        in_specs=[
            pl.BlockSpec((_BM, _RANK), lambda i: (i, 0)),
            pl.BlockSpec((_RANK, _DIM), lambda i: (0, 0)),
        ],
        out_specs=pl.BlockSpec((_BM, _DIM), lambda i: (i, 0)),
        out_shape=jax.ShapeDtypeStruct((_TOTAL, _DIM), jnp.float32),
        compiler_params=pltpu.CompilerParams(
            dimension_semantics=("arbitrary",),
        ),
    )(emb, B)


def kernel(token_ids, A, B):
    idx = token_ids.reshape(_NW, _NCH, _CHUNK).astype(jnp.int32)
    emb = _sc_gather_call(idx, A)
    out = _tc_matmul(emb, B)
    return out.reshape(_BATCH, _HIST, _DIM)


# speedup vs baseline: 25.1809x; 2.2631x over previous
import functools

import jax
import jax.numpy as jnp
from jax import lax
from jax.experimental import pallas as pl
from jax.experimental.pallas import tpu as pltpu
from jax.experimental.pallas import tpu_sc as plsc

_VOCAB = 1000000
_RANK = 32
_DIM = 64
_BATCH = 16384
_HIST = 50
_TOTAL = _BATCH * _HIST

_NC = 2
_NS = 16
_NW = _NC * _NS
_PER_W = _TOTAL // _NW
_CHUNK = 128
_NCH = _PER_W // _CHUNK
_NPACK = 128 // _RANK


def _sc_gather(idx_hbm, table_hbm, out_hbm, idx_v, rows_v, sem):
    wid = lax.axis_index("s") * _NC + lax.axis_index("c")
    pltpu.sync_copy(idx_hbm.at[wid], idx_v)

    def body(j, carry):
        pltpu.async_copy(table_hbm.at[idx_v.at[j]], rows_v, sem).wait()
        pltpu.sync_copy(rows_v, out_hbm.at[wid * _NCH + j])
        return carry

    lax.fori_loop(0, _NCH, body, 0, unroll=False)


_sc_gather_call = functools.partial(
    pl.kernel,
    out_type=jax.ShapeDtypeStruct((_NW * _NCH, _CHUNK, _RANK), jnp.float32),
    mesh=plsc.VectorSubcoreMesh(core_axis_name="c", subcore_axis_name="s"),
    scratch_types=[
        pltpu.VMEM((_NCH, _CHUNK), jnp.int32),
        pltpu.VMEM((_CHUNK, _RANK), jnp.float32),
        pltpu.SemaphoreType.DMA,
    ],
    compiler_params=pltpu.CompilerParams(use_tc_tiling_on_sc=False),
)(_sc_gather)


_GP = _BATCH // _NPACK


def _tc_matmul_body(emb_ref, bp_ref, out_ref):
    t = lax.dot_general(
        bp_ref[...], emb_ref[0],
        ((( 0,), (1,)), ((), ())),
        preferred_element_type=jnp.float32,
    )
    for a in range(_NPACK):
        out_ref[0, :, a * _GP:(a + 1) * _GP] = t[a * _DIM:(a + 1) * _DIM, :]


def _tc_matmul(emb_p, Bp):
    return pl.pallas_call(
        _tc_matmul_body,
        grid=(_HIST,),
        in_specs=[
            pl.BlockSpec((1, _GP, 128), lambda h: (h, 0, 0)),
            pl.BlockSpec((128, _NPACK * _DIM), lambda h: (0, 0)),
        ],
        out_specs=pl.BlockSpec((1, _DIM, _BATCH), lambda h: (h, 0, 0)),
        out_shape=jax.ShapeDtypeStruct((_HIST, _DIM, _BATCH), jnp.float32),
        compiler_params=pltpu.CompilerParams(
            dimension_semantics=("arbitrary",),
        ),
    )(emb_p, Bp)


def kernel(token_ids, A, B):
    tokT = token_ids.T.reshape(_HIST, _NPACK, 128, 32)
    idx = tokT.transpose(0, 2, 3, 1).reshape(_NW, _NCH, _CHUNK)
    idx = idx.astype(jnp.int32)
    emb = _sc_gather_call(idx, A)
    emb_p = emb.reshape(_HIST, _GP, 128)
    eye = jnp.eye(_NPACK, dtype=B.dtype)
    Bp = (eye[:, None, :, None] * B[None, :, None, :]).reshape(
        _NPACK * _RANK, _NPACK * _DIM
    )
    out_t = _tc_matmul(emb_p, Bp)
    return out_t.transpose(2, 0, 1)
